# Initial kernel scaffold; baseline (speedup 1.0000x reference)
import jax
import jax.numpy as jnp
from jax.experimental import pallas as pl


def kernel(input_, weight):
    raise NotImplementedError("write your pallas kernel here")


# speedup vs baseline: 1.5726x; 1.5726x over previous
import functools

import jax
import jax.numpy as jnp
from jax import lax
from jax.experimental import pallas as pl
from jax.experimental.pallas import tpu as pltpu
from jax.experimental.pallas import tpu_sc as plsc

NUM_CORES = 2
NUM_SUBCORES = 16
NW = NUM_CORES * NUM_SUBCORES

CHUNK = 128


@functools.partial(jax.jit, static_argnames=("b_per_w", "n_chunks"))
def _gather_rows(idx_flat, weight, *, b_per_w, n_chunks):
    B = idx_flat.shape[0]
    D = weight.shape[1]
    mesh = plsc.VectorSubcoreMesh(core_axis_name="c", subcore_axis_name="s")

    @functools.partial(
        pl.kernel,
        mesh=mesh,
        out_type=jax.ShapeDtypeStruct((B, D), jnp.float32),
        scratch_types=[
            pltpu.VMEM((CHUNK,), jnp.int32),
            pltpu.VMEM((CHUNK, D), jnp.float32),
            pltpu.SemaphoreType.DMA,
        ],
        compiler_params=pltpu.CompilerParams(use_tc_tiling_on_sc=False),
    )
    def k(idx_hbm, table_hbm, out_hbm, idx_v, rows_v, sem):
        wid = lax.axis_index("s") * NUM_CORES + lax.axis_index("c")
        base = wid * b_per_w

        def body(j, carry):
            off = base + j * CHUNK
            pltpu.sync_copy(idx_hbm.at[pl.ds(off, CHUNK)], idx_v)
            pltpu.async_copy(table_hbm.at[idx_v], rows_v, sem).wait()
            pltpu.sync_copy(rows_v, out_hbm.at[pl.ds(off, CHUNK)])
            return carry

        lax.fori_loop(0, n_chunks, body, 0)

    return k(idx_flat, weight)


def kernel(input_, weight):
    B = input_.shape[0] * input_.shape[1]
    idx_flat = input_.reshape(B).astype(jnp.int32)
    b_per_w = B // NW
    n_chunks = b_per_w // CHUNK
    out = _gather_rows(idx_flat, weight, b_per_w=b_per_w, n_chunks=n_chunks)
    return out.reshape(input_.shape[0], input_.shape[1], weight.shape[1])


# speedup vs baseline: 1.8738x; 1.1915x over previous
import functools

import jax
import jax.numpy as jnp
from jax import lax
from jax.experimental import pallas as pl
from jax.experimental.pallas import tpu as pltpu
from jax.experimental.pallas import tpu_sc as plsc

NUM_CORES = 2
NUM_SUBCORES = 16
NW = NUM_CORES * NUM_SUBCORES

CHUNK = 512
NBUF = 2


@functools.partial(jax.jit, static_argnames=("b_per_w", "n_groups"))
def _gather_rows(idx_flat, weight, *, b_per_w, n_groups):
    B = idx_flat.shape[0]
    D = weight.shape[1]
    mesh = plsc.VectorSubcoreMesh(core_axis_name="c", subcore_axis_name="s")

    @functools.partial(
        pl.kernel,
        mesh=mesh,
        out_type=jax.ShapeDtypeStruct((B, D), jnp.float32),
        scratch_types=[
            pltpu.VMEM((NBUF, CHUNK), jnp.int32),
            pltpu.VMEM((NBUF, CHUNK, D), jnp.float32),
            pltpu.SemaphoreType.DMA,
            pltpu.SemaphoreType.DMA,
            pltpu.SemaphoreType.DMA,
            pltpu.SemaphoreType.DMA,
        ],
        compiler_params=pltpu.CompilerParams(use_tc_tiling_on_sc=False),
    )
    def k(idx_hbm, table_hbm, out_hbm, idx_v, rows_v, g0, g1, o0, o1):
        gsems = (g0, g1)
        osems = (o0, o1)
        wid = lax.axis_index("s") * NUM_CORES + lax.axis_index("c")
        base = wid * b_per_w

        def group(g, carry):
            descs = []
            for b in range(NBUF):
                off = base + (g * NBUF + b) * CHUNK

                @pl.when(g > 0)
                def _drain():
                    pltpu.make_async_copy(
                        rows_v.at[b],
                        out_hbm.at[pl.ds(off - NBUF * CHUNK, CHUNK)],
                        osems[b],
                    ).wait()

                pltpu.sync_copy(idx_hbm.at[pl.ds(off, CHUNK)], idx_v.at[b])
                descs.append(
                    pltpu.async_copy(table_hbm.at[idx_v.at[b]], rows_v.at[b],
                                     gsems[b]))
            for b in range(NBUF):
                off = base + (g * NBUF + b) * CHUNK
                descs[b].wait()
                pltpu.async_copy(rows_v.at[b], out_hbm.at[pl.ds(off, CHUNK)],
                                 osems[b])
            return carry

        lax.fori_loop(0, n_groups, group, 0)
        for b in range(NBUF):
            off = base + ((n_groups - 1) * NBUF + b) * CHUNK
            pltpu.make_async_copy(
                rows_v.at[b], out_hbm.at[pl.ds(off, CHUNK)], osems[b]).wait()

    return k(idx_flat, weight)


def kernel(input_, weight):
    B = input_.shape[0] * input_.shape[1]
    idx_flat = input_.reshape(B).astype(jnp.int32)
    b_per_w = B // NW
    n_groups = b_per_w // (CHUNK * NBUF)
    out = _gather_rows(idx_flat, weight, b_per_w=b_per_w, n_groups=n_groups)
    return out.reshape(input_.shape[0], input_.shape[1], weight.shape[1])
